# baseline (speedup 1.0000x reference)
import functools

import jax
import jax.numpy as jnp
from jax import lax
from jax.experimental import pallas as pl
from jax.experimental.pallas import tpu as pltpu
from jax.experimental.pallas import tpu_sc as plsc

N = 10000
D = 128
E = 320000
LIVE = E + N
NC, NS = 2, 16
NW = NC * NS
CHUNK = 128
NCHUNK = 88
EPT = NCHUNK * CHUNK
E_PAD = EPT * NW
ACC_ROWS = 10112
ZROWS = ACC_ROWS // NS
ZFULL = ZROWS // CHUNK
ZTAIL = ZROWS - ZFULL * CHUNK
CW = 16
NBUF = 3
NIS = 3
NID = 4

_sc_mesh = plsc.VectorSubcoreMesh(core_axis_name="c", subcore_axis_name="s")


def _live_chunks(wid):
    n = (LIVE - wid * EPT + CHUNK - 1) // CHUNK
    return jnp.clip(n, 0, NCHUNK)


@functools.partial(
    pl.kernel,
    out_type=jax.ShapeDtypeStruct((NC, ACC_ROWS, CW), jnp.float32),
    mesh=_sc_mesh,
    scratch_types=[
        pltpu.VMEM_SHARED((ACC_ROWS, CW), jnp.float32),
        pltpu.VMEM((NCHUNK, CHUNK), jnp.int32),
        pltpu.VMEM((CHUNK, CW), jnp.float32),
    ],
)
def _count_kernel(dst_hbm, out_hbm, acc_sh, idx_v, ones_v):
    cid = lax.axis_index("c")
    sid = lax.axis_index("s")
    wid = cid * NS + sid

    def _fill(val):
        def body(i, _):
            ones_v[i, pl.ds(0, 16)] = jnp.zeros((16,), jnp.float32) + val
            return 0
        lax.fori_loop(0, CHUNK, body, 0)

    _fill(0.0)

    def _zero(k, _):
        pltpu.sync_copy(ones_v, acc_sh.at[pl.ds(sid * ZROWS + k * CHUNK, CHUNK)])
        return 0

    lax.fori_loop(0, ZFULL, _zero, 0)
    pltpu.sync_copy(ones_v.at[pl.ds(0, ZTAIL)],
                    acc_sh.at[pl.ds(sid * ZROWS + ZFULL * CHUNK, ZTAIL)])
    _fill(1.0)
    plsc.subcore_barrier()

    pltpu.sync_copy(dst_hbm.at[pl.ds(wid * NCHUNK, NCHUNK)], idx_v)

    def _scat(j, _):
        pltpu.sync_copy(ones_v, acc_sh.at[idx_v.at[j]], add=True)
        return 0

    lax.fori_loop(0, _live_chunks(wid), _scat, 0)
    plsc.subcore_barrier()
    pltpu.sync_copy(acc_sh.at[pl.ds(sid * ZROWS, ZROWS)],
                    out_hbm.at[cid, pl.ds(sid * ZROWS, ZROWS)])


@functools.partial(
    pl.kernel,
    out_type=jax.ShapeDtypeStruct((NC, ACC_ROWS, D), jnp.float32),
    mesh=_sc_mesh,
    scratch_types=[
        pltpu.VMEM_SHARED((ACC_ROWS, D), jnp.float32),
        pltpu.VMEM((NIS, CHUNK), jnp.int32),
        pltpu.VMEM((NID, CHUNK), jnp.int32),
        pltpu.VMEM((NBUF, CHUNK, D), jnp.float32),
        pltpu.SemaphoreType.DMA,
        pltpu.SemaphoreType.DMA,
        pltpu.SemaphoreType.DMA,
    ],
)
def _prop_kernel(hp_hbm, src_hbm, dst_hbm, out_hbm, acc_sh, sidx_v, didx_v,
                 rows_v, isem, gsem, ssem):
    cid = lax.axis_index("c")
    sid = lax.axis_index("s")
    wid = cid * NS + sid

    def _zfill(i, _):
        rows_v[0, i // (D // 16), pl.ds((i % (D // 16)) * 16, 16)] = jnp.zeros(
            (16,), jnp.float32)
        return 0

    lax.fori_loop(0, CHUNK * D // 16, _zfill, 0)

    def _zero(k, _):
        pltpu.sync_copy(rows_v.at[0],
                        acc_sh.at[pl.ds(sid * ZROWS + k * CHUNK, CHUNK)])
        return 0

    lax.fori_loop(0, ZFULL, _zero, 0)
    pltpu.sync_copy(rows_v.at[0, pl.ds(0, ZTAIL)],
                    acc_sh.at[pl.ds(sid * ZROWS + ZFULL * CHUNK, ZTAIL)])
    plsc.subcore_barrier()

    nlive = _live_chunks(wid)

    def _start_i(jj):
        base = (wid * NCHUNK + jj) * CHUNK
        pltpu.async_copy(src_hbm.at[pl.ds(base, CHUNK)], sidx_v.at[jj % NIS],
                         isem)
        pltpu.async_copy(dst_hbm.at[pl.ds(base, CHUNK)], didx_v.at[jj % NID],
                         isem)

    def _wait_i(jj):
        base = (wid * NCHUNK + jj) * CHUNK
        pltpu.make_async_copy(src_hbm.at[pl.ds(base, CHUNK)],
                              sidx_v.at[jj % NIS], isem).wait()
        pltpu.make_async_copy(dst_hbm.at[pl.ds(base, CHUNK)],
                              didx_v.at[jj % NID], isem).wait()

    def _start_g(jj):
        pltpu.async_copy(hp_hbm.at[sidx_v.at[jj % NIS]], rows_v.at[jj % NBUF],
                         gsem)

    def _wait_g(jj):
        pltpu.make_async_copy(hp_hbm.at[sidx_v.at[jj % NIS]],
                              rows_v.at[jj % NBUF], gsem).wait()

    def _start_s(jj):
        pltpu.async_copy(rows_v.at[jj % NBUF], acc_sh.at[didx_v.at[jj % NID]],
                         ssem, add=True)

    def _wait_s(jj):
        pltpu.make_async_copy(rows_v.at[jj % NBUF],
                              acc_sh.at[didx_v.at[jj % NID]], ssem).wait()

    for k in range(NBUF):
        @pl.when(k < nlive)
        def _():
            _start_i(k)

    for k in range(NBUF - 1):
        @pl.when(k < nlive)
        def _():
            _wait_i(k)
            _start_g(k)

    def _edge(j, _):
        _wait_g(j)
        _start_s(j)

        @pl.when(j >= 1)
        def _():
            _wait_s(j - 1)

        @pl.when(j + 2 < nlive)
        def _():
            _wait_i(j + 2)
            _start_g(j + 2)

        @pl.when(j + 3 < nlive)
        def _():
            _start_i(j + 3)

        return 0

    lax.fori_loop(0, nlive, _edge, 0)

    @pl.when(nlive >= 1)
    def _():
        _wait_s(nlive - 1)

    plsc.subcore_barrier()
    pltpu.sync_copy(acc_sh.at[pl.ds(sid * ZROWS, ZROWS)],
                    out_hbm.at[cid, pl.ds(sid * ZROWS, ZROWS)])


BLK = 1000
GRID = N // BLK


def _disb(cnt):
    deg = cnt[0, :, 0:1] + cnt[1, :, 0:1]
    return jnp.broadcast_to(lax.rsqrt(deg), (BLK, D))


def _in_body(cnt_ref, x_ref, w_ref, hp_ref):
    h = jnp.dot(x_ref[...], w_ref[...], preferred_element_type=jnp.float32)
    hp_ref[...] = _disb(cnt_ref[...]) * h


_in_call = pl.pallas_call(
    _in_body,
    grid=(GRID,),
    in_specs=[
        pl.BlockSpec((NC, BLK, CW), lambda i: (0, i, 0)),
        pl.BlockSpec((BLK, D), lambda i: (i, 0)),
        pl.BlockSpec((D, D), lambda i: (0, 0)),
    ],
    out_specs=pl.BlockSpec((BLK, D), lambda i: (i, 0)),
    out_shape=jax.ShapeDtypeStruct((N, D), jnp.float32),
)


def _mid_body(cnt_ref, p_ref, b_ref, w_ref, hn_ref):
    p = p_ref[...]
    disb = _disb(cnt_ref[...])
    act = jnp.maximum(disb * (p[0] + p[1]) + b_ref[...], 0.0)
    hn_ref[...] = disb * jnp.dot(act, w_ref[...],
                                 preferred_element_type=jnp.float32)


def _mid_act_body(cnt_ref, p_ref, b_ref, w_ref, hn_ref, act_ref):
    p = p_ref[...]
    disb = _disb(cnt_ref[...])
    act = jnp.maximum(disb * (p[0] + p[1]) + b_ref[...], 0.0)
    hn_ref[...] = disb * jnp.dot(act, w_ref[...],
                                 preferred_element_type=jnp.float32)
    act_ref[...] = act


_mid_specs = dict(
    grid=(GRID,),
    in_specs=[
        pl.BlockSpec((NC, BLK, CW), lambda i: (0, i, 0)),
        pl.BlockSpec((NC, BLK, D), lambda i: (0, i, 0)),
        pl.BlockSpec((1, D), lambda i: (0, 0)),
        pl.BlockSpec((D, D), lambda i: (0, 0)),
    ],
)

_mid_call = pl.pallas_call(
    _mid_body,
    out_specs=pl.BlockSpec((BLK, D), lambda i: (i, 0)),
    out_shape=jax.ShapeDtypeStruct((N, D), jnp.float32),
    **_mid_specs,
)

_mid_act_call = pl.pallas_call(
    _mid_act_body,
    out_specs=[
        pl.BlockSpec((BLK, D), lambda i: (i, 0)),
        pl.BlockSpec((BLK, D), lambda i: (i, 0)),
    ],
    out_shape=[
        jax.ShapeDtypeStruct((N, D), jnp.float32),
        jax.ShapeDtypeStruct((N, D), jnp.float32),
    ],
    **_mid_specs,
)


def _out_body(cnt_ref, p_ref, b_ref, xh_ref):
    p = p_ref[...]
    xh_ref[...] = _disb(cnt_ref[...]) * (p[0] + p[1]) + b_ref[...]


_out_call = pl.pallas_call(
    _out_body,
    grid=(GRID,),
    in_specs=[
        pl.BlockSpec((NC, BLK, CW), lambda i: (0, i, 0)),
        pl.BlockSpec((NC, BLK, D), lambda i: (0, i, 0)),
        pl.BlockSpec((1, D), lambda i: (0, 0)),
    ],
    out_specs=pl.BlockSpec((BLK, D), lambda i: (i, 0)),
    out_shape=jax.ShapeDtypeStruct((N, D), jnp.float32),
)


def kernel(x, edge_index, W1, b1, W2, b2, W3, b3, W4, b4):
    iota = jnp.arange(N, dtype=jnp.int32)
    pad = E_PAD - LIVE
    ext_src = jnp.concatenate(
        [edge_index[0], iota, jnp.zeros((pad,), jnp.int32)])
    ext_dst = jnp.concatenate(
        [edge_index[1], iota, jnp.full((pad,), N, jnp.int32)])

    cnt = _count_kernel(ext_dst.reshape(NW * NCHUNK, CHUNK))
    hp1 = _in_call(cnt, x, W1)
    p1 = _prop_kernel(hp1, ext_src, ext_dst)
    hp2 = _mid_call(cnt, p1, b1.reshape(1, D), W2)
    p2 = _prop_kernel(hp2, ext_src, ext_dst)
    hp3, z = _mid_act_call(cnt, p2, b2.reshape(1, D), W3)
    p3 = _prop_kernel(hp3, ext_src, ext_dst)
    hp4 = _mid_call(cnt, p3, b3.reshape(1, D), W4)
    p4 = _prop_kernel(hp4, ext_src, ext_dst)
    x_hat = _out_call(cnt, p4, b4.reshape(1, D))
    return (x_hat, z)


# speedup vs baseline: 1.0007x; 1.0007x over previous
import functools

import jax
import jax.numpy as jnp
from jax import lax
from jax.experimental import pallas as pl
from jax.experimental.pallas import tpu as pltpu
from jax.experimental.pallas import tpu_sc as plsc

N = 10000
D = 128
E = 320000
LIVE = E + N
NC, NS = 2, 16
NW = NC * NS
CHUNK = 128
NCHUNK = 88
EPT = NCHUNK * CHUNK
E_PAD = EPT * NW
ACC_ROWS = 10112
ZROWS = ACC_ROWS // NS
ZFULL = ZROWS // CHUNK
ZTAIL = ZROWS - ZFULL * CHUNK
CW = 16
NBUF = 3
NIS = 3
NID = 4

_sc_mesh = plsc.VectorSubcoreMesh(core_axis_name="c", subcore_axis_name="s")


def _live_chunks(wid):
    n = (LIVE - wid * EPT + CHUNK - 1) // CHUNK
    return jnp.clip(n, 0, NCHUNK)


@functools.partial(
    pl.kernel,
    out_type=jax.ShapeDtypeStruct((NC, ACC_ROWS, CW), jnp.float32),
    mesh=_sc_mesh,
    scratch_types=[
        pltpu.VMEM_SHARED((ACC_ROWS, CW), jnp.float32),
        pltpu.VMEM((NCHUNK, CHUNK), jnp.int32),
        pltpu.VMEM((CHUNK, CW), jnp.float32),
    ],
)
def _count_kernel(dst_hbm, out_hbm, acc_sh, idx_v, ones_v):
    cid = lax.axis_index("c")
    sid = lax.axis_index("s")
    wid = cid * NS + sid

    def _fill(val):
        def body(i, _):
            ones_v[i, pl.ds(0, 16)] = jnp.zeros((16,), jnp.float32) + val
            return 0
        lax.fori_loop(0, CHUNK, body, 0)

    _fill(0.0)

    def _zero(k, _):
        pltpu.sync_copy(ones_v, acc_sh.at[pl.ds(sid * ZROWS + k * CHUNK, CHUNK)])
        return 0

    lax.fori_loop(0, ZFULL, _zero, 0)
    pltpu.sync_copy(ones_v.at[pl.ds(0, ZTAIL)],
                    acc_sh.at[pl.ds(sid * ZROWS + ZFULL * CHUNK, ZTAIL)])
    _fill(1.0)
    plsc.subcore_barrier()

    pltpu.sync_copy(dst_hbm.at[pl.ds(wid * NCHUNK, NCHUNK)], idx_v)

    def _scat(j, _):
        pltpu.sync_copy(ones_v, acc_sh.at[idx_v.at[j]], add=True)
        return 0

    lax.fori_loop(0, _live_chunks(wid), _scat, 0)
    plsc.subcore_barrier()
    pltpu.sync_copy(acc_sh.at[pl.ds(sid * ZROWS, ZROWS)],
                    out_hbm.at[cid, pl.ds(sid * ZROWS, ZROWS)])


@functools.partial(
    pl.kernel,
    out_type=jax.ShapeDtypeStruct((NC, ACC_ROWS, D), jnp.float32),
    mesh=_sc_mesh,
    scratch_types=[
        pltpu.VMEM_SHARED((ACC_ROWS, D), jnp.float32),
        pltpu.VMEM((NIS, CHUNK), jnp.int32),
        pltpu.VMEM((NID, CHUNK), jnp.int32),
        pltpu.VMEM((NBUF, CHUNK, D), jnp.float32),
        pltpu.SemaphoreType.DMA,
        pltpu.SemaphoreType.DMA,
        pltpu.SemaphoreType.DMA,
    ],
)
def _prop_kernel(hp_hbm, src_hbm, dst_hbm, out_hbm, acc_sh, sidx_v, didx_v,
                 rows_v, isem, gsem, ssem):
    cid = lax.axis_index("c")
    sid = lax.axis_index("s")
    wid = cid * NS + sid

    nlive = _live_chunks(wid)

    def _start_i(jj):
        base = (wid * NCHUNK + jj) * CHUNK
        pltpu.async_copy(src_hbm.at[pl.ds(base, CHUNK)], sidx_v.at[jj % NIS],
                         isem)
        pltpu.async_copy(dst_hbm.at[pl.ds(base, CHUNK)], didx_v.at[jj % NID],
                         isem)

    def _wait_i(jj):
        base = (wid * NCHUNK + jj) * CHUNK
        pltpu.make_async_copy(src_hbm.at[pl.ds(base, CHUNK)],
                              sidx_v.at[jj % NIS], isem).wait()
        pltpu.make_async_copy(dst_hbm.at[pl.ds(base, CHUNK)],
                              didx_v.at[jj % NID], isem).wait()

    def _start_g(jj):
        pltpu.async_copy(hp_hbm.at[sidx_v.at[jj % NIS]], rows_v.at[jj % NBUF],
                         gsem)

    def _wait_g(jj):
        pltpu.make_async_copy(hp_hbm.at[sidx_v.at[jj % NIS]],
                              rows_v.at[jj % NBUF], gsem).wait()

    def _start_s(jj):
        pltpu.async_copy(rows_v.at[jj % NBUF], acc_sh.at[didx_v.at[jj % NID]],
                         ssem, add=True)

    def _wait_s(jj):
        pltpu.make_async_copy(rows_v.at[jj % NBUF],
                              acc_sh.at[didx_v.at[jj % NID]], ssem).wait()

    for k in range(NBUF):
        @pl.when(k < nlive)
        def _():
            _start_i(k)

    def _zfill(i, _):
        rows_v[NBUF - 1, i // (D // 16),
               pl.ds((i % (D // 16)) * 16, 16)] = jnp.zeros((16,), jnp.float32)
        return 0

    lax.fori_loop(0, CHUNK * D // 16, _zfill, 0)

    def _zero(k, _):
        pltpu.sync_copy(rows_v.at[NBUF - 1],
                        acc_sh.at[pl.ds(sid * ZROWS + k * CHUNK, CHUNK)])
        return 0

    lax.fori_loop(0, ZFULL, _zero, 0)
    pltpu.sync_copy(rows_v.at[NBUF - 1, pl.ds(0, ZTAIL)],
                    acc_sh.at[pl.ds(sid * ZROWS + ZFULL * CHUNK, ZTAIL)])

    for k in range(NBUF - 1):
        @pl.when(k < nlive)
        def _():
            _wait_i(k)
            _start_g(k)

    plsc.subcore_barrier()

    def _edge(j, _):
        _wait_g(j)
        _start_s(j)

        @pl.when(j >= 1)
        def _():
            _wait_s(j - 1)

        @pl.when(j + 2 < nlive)
        def _():
            _wait_i(j + 2)
            _start_g(j + 2)

        @pl.when(j + 3 < nlive)
        def _():
            _start_i(j + 3)

        return 0

    lax.fori_loop(0, nlive, _edge, 0)

    @pl.when(nlive >= 1)
    def _():
        _wait_s(nlive - 1)

    plsc.subcore_barrier()
    pltpu.sync_copy(acc_sh.at[pl.ds(sid * ZROWS, ZROWS)],
                    out_hbm.at[cid, pl.ds(sid * ZROWS, ZROWS)])


BLK = 1000
GRID = N // BLK


def _disb(cnt):
    deg = cnt[0, :, 0:1] + cnt[1, :, 0:1]
    return jnp.broadcast_to(lax.rsqrt(deg), (BLK, D))


def _in_body(cnt_ref, x_ref, w_ref, hp_ref):
    h = jnp.dot(x_ref[...], w_ref[...], preferred_element_type=jnp.float32)
    hp_ref[...] = _disb(cnt_ref[...]) * h


_in_call = pl.pallas_call(
    _in_body,
    grid=(GRID,),
    in_specs=[
        pl.BlockSpec((NC, BLK, CW), lambda i: (0, i, 0)),
        pl.BlockSpec((BLK, D), lambda i: (i, 0)),
        pl.BlockSpec((D, D), lambda i: (0, 0)),
    ],
    out_specs=pl.BlockSpec((BLK, D), lambda i: (i, 0)),
    out_shape=jax.ShapeDtypeStruct((N, D), jnp.float32),
)


def _mid_body(cnt_ref, p_ref, b_ref, w_ref, hn_ref):
    p = p_ref[...]
    disb = _disb(cnt_ref[...])
    act = jnp.maximum(disb * (p[0] + p[1]) + b_ref[...], 0.0)
    hn_ref[...] = disb * jnp.dot(act, w_ref[...],
                                 preferred_element_type=jnp.float32)


def _mid_act_body(cnt_ref, p_ref, b_ref, w_ref, hn_ref, act_ref):
    p = p_ref[...]
    disb = _disb(cnt_ref[...])
    act = jnp.maximum(disb * (p[0] + p[1]) + b_ref[...], 0.0)
    hn_ref[...] = disb * jnp.dot(act, w_ref[...],
                                 preferred_element_type=jnp.float32)
    act_ref[...] = act


_mid_specs = dict(
    grid=(GRID,),
    in_specs=[
        pl.BlockSpec((NC, BLK, CW), lambda i: (0, i, 0)),
        pl.BlockSpec((NC, BLK, D), lambda i: (0, i, 0)),
        pl.BlockSpec((1, D), lambda i: (0, 0)),
        pl.BlockSpec((D, D), lambda i: (0, 0)),
    ],
)

_mid_call = pl.pallas_call(
    _mid_body,
    out_specs=pl.BlockSpec((BLK, D), lambda i: (i, 0)),
    out_shape=jax.ShapeDtypeStruct((N, D), jnp.float32),
    **_mid_specs,
)

_mid_act_call = pl.pallas_call(
    _mid_act_body,
    out_specs=[
        pl.BlockSpec((BLK, D), lambda i: (i, 0)),
        pl.BlockSpec((BLK, D), lambda i: (i, 0)),
    ],
    out_shape=[
        jax.ShapeDtypeStruct((N, D), jnp.float32),
        jax.ShapeDtypeStruct((N, D), jnp.float32),
    ],
    **_mid_specs,
)


def _out_body(cnt_ref, p_ref, b_ref, xh_ref):
    p = p_ref[...]
    xh_ref[...] = _disb(cnt_ref[...]) * (p[0] + p[1]) + b_ref[...]


_out_call = pl.pallas_call(
    _out_body,
    grid=(GRID,),
    in_specs=[
        pl.BlockSpec((NC, BLK, CW), lambda i: (0, i, 0)),
        pl.BlockSpec((NC, BLK, D), lambda i: (0, i, 0)),
        pl.BlockSpec((1, D), lambda i: (0, 0)),
    ],
    out_specs=pl.BlockSpec((BLK, D), lambda i: (i, 0)),
    out_shape=jax.ShapeDtypeStruct((N, D), jnp.float32),
)


def kernel(x, edge_index, W1, b1, W2, b2, W3, b3, W4, b4):
    iota = jnp.arange(N, dtype=jnp.int32)
    pad = E_PAD - LIVE
    ext_src = jnp.concatenate(
        [edge_index[0], iota, jnp.zeros((pad,), jnp.int32)])
    ext_dst = jnp.concatenate(
        [edge_index[1], iota, jnp.full((pad,), N, jnp.int32)])

    cnt = _count_kernel(ext_dst.reshape(NW * NCHUNK, CHUNK))
    hp1 = _in_call(cnt, x, W1)
    p1 = _prop_kernel(hp1, ext_src, ext_dst)
    hp2 = _mid_call(cnt, p1, b1.reshape(1, D), W2)
    p2 = _prop_kernel(hp2, ext_src, ext_dst)
    hp3, z = _mid_act_call(cnt, p2, b2.reshape(1, D), W3)
    p3 = _prop_kernel(hp3, ext_src, ext_dst)
    hp4 = _mid_call(cnt, p3, b3.reshape(1, D), W4)
    p4 = _prop_kernel(hp4, ext_src, ext_dst)
    x_hat = _out_call(cnt, p4, b4.reshape(1, D))
    return (x_hat, z)


# speedup vs baseline: 1.0121x; 1.0113x over previous
import functools

import jax
import jax.numpy as jnp
from jax import lax
from jax.experimental import pallas as pl
from jax.experimental.pallas import tpu as pltpu
from jax.experimental.pallas import tpu_sc as plsc

N = 10000
D = 128
E = 320000
LIVE = E + N
NC, NS = 2, 16
NW = NC * NS
CHUNK = 128
NCHUNK = 88
EPT = NCHUNK * CHUNK
E_PAD = EPT * NW
ACC_ROWS = 10112
ZROWS = ACC_ROWS // NS
ZFULL = ZROWS // CHUNK
ZTAIL = ZROWS - ZFULL * CHUNK
CW = 16
NBUF = 3
NIS = 3
NID = 4

_sc_mesh = plsc.VectorSubcoreMesh(core_axis_name="c", subcore_axis_name="s")


def _live_chunks(wid):
    n = (LIVE - wid * EPT + CHUNK - 1) // CHUNK
    return jnp.clip(n, 0, NCHUNK)


@functools.partial(
    pl.kernel,
    out_type=jax.ShapeDtypeStruct((NC, ACC_ROWS, CW), jnp.float32),
    mesh=_sc_mesh,
    scratch_types=[
        pltpu.VMEM_SHARED((ACC_ROWS, CW), jnp.float32),
        pltpu.VMEM((NCHUNK, CHUNK), jnp.int32),
        pltpu.VMEM((CHUNK, CW), jnp.float32),
        pltpu.SemaphoreType.DMA,
    ],
)
def _count_kernel(dst_hbm, out_hbm, acc_sh, idx_v, ones_v, csem):
    cid = lax.axis_index("c")
    sid = lax.axis_index("s")
    wid = cid * NS + sid

    def _fill(val):
        def body(i, _):
            ones_v[i, pl.ds(0, 16)] = jnp.zeros((16,), jnp.float32) + val
            return 0
        lax.fori_loop(0, CHUNK, body, 0)

    _fill(0.0)

    def _zero(k, _):
        pltpu.sync_copy(ones_v, acc_sh.at[pl.ds(sid * ZROWS + k * CHUNK, CHUNK)])
        return 0

    lax.fori_loop(0, ZFULL, _zero, 0)
    pltpu.sync_copy(ones_v.at[pl.ds(0, ZTAIL)],
                    acc_sh.at[pl.ds(sid * ZROWS + ZFULL * CHUNK, ZTAIL)])
    _fill(1.0)
    plsc.subcore_barrier()

    pltpu.sync_copy(dst_hbm.at[pl.ds(wid * NCHUNK, NCHUNK)], idx_v)

    nlive = _live_chunks(wid)

    def _scat(j, _):
        pltpu.async_copy(ones_v, acc_sh.at[idx_v.at[j]], csem, add=True)

        @pl.when(j >= 16)
        def _():
            pltpu.make_async_copy(ones_v, acc_sh.at[idx_v.at[0]], csem).wait()

        return 0

    lax.fori_loop(0, nlive, _scat, 0)

    def _cdrain(j, _):
        pltpu.make_async_copy(ones_v, acc_sh.at[idx_v.at[0]], csem).wait()
        return 0

    lax.fori_loop(0, jnp.minimum(nlive, 16), _cdrain, 0)
    plsc.subcore_barrier()
    pltpu.sync_copy(acc_sh.at[pl.ds(sid * ZROWS, ZROWS)],
                    out_hbm.at[cid, pl.ds(sid * ZROWS, ZROWS)])


@functools.partial(
    pl.kernel,
    out_type=jax.ShapeDtypeStruct((NC, ACC_ROWS, D), jnp.float32),
    mesh=_sc_mesh,
    scratch_types=[
        pltpu.VMEM_SHARED((ACC_ROWS, D), jnp.float32),
        pltpu.VMEM((NIS, CHUNK), jnp.int32),
        pltpu.VMEM((NID, CHUNK), jnp.int32),
        pltpu.VMEM((NBUF, CHUNK, D), jnp.float32),
        pltpu.SemaphoreType.DMA,
        pltpu.SemaphoreType.DMA,
        pltpu.SemaphoreType.DMA,
    ],
)
def _prop_kernel(hp_hbm, src_hbm, dst_hbm, out_hbm, acc_sh, sidx_v, didx_v,
                 rows_v, isem, gsem, ssem):
    cid = lax.axis_index("c")
    sid = lax.axis_index("s")
    wid = cid * NS + sid

    nlive = _live_chunks(wid)

    def _start_i(jj):
        base = (wid * NCHUNK + jj) * CHUNK
        pltpu.async_copy(src_hbm.at[pl.ds(base, CHUNK)], sidx_v.at[jj % NIS],
                         isem)
        pltpu.async_copy(dst_hbm.at[pl.ds(base, CHUNK)], didx_v.at[jj % NID],
                         isem)

    def _wait_i(jj):
        base = (wid * NCHUNK + jj) * CHUNK
        pltpu.make_async_copy(src_hbm.at[pl.ds(base, CHUNK)],
                              sidx_v.at[jj % NIS], isem).wait()
        pltpu.make_async_copy(dst_hbm.at[pl.ds(base, CHUNK)],
                              didx_v.at[jj % NID], isem).wait()

    def _start_g(jj):
        pltpu.async_copy(hp_hbm.at[sidx_v.at[jj % NIS]], rows_v.at[jj % NBUF],
                         gsem)

    def _wait_g(jj):
        pltpu.make_async_copy(hp_hbm.at[sidx_v.at[jj % NIS]],
                              rows_v.at[jj % NBUF], gsem).wait()

    def _start_s(jj):
        pltpu.async_copy(rows_v.at[jj % NBUF], acc_sh.at[didx_v.at[jj % NID]],
                         ssem, add=True)

    def _wait_s(jj):
        pltpu.make_async_copy(rows_v.at[jj % NBUF],
                              acc_sh.at[didx_v.at[jj % NID]], ssem).wait()

    for k in range(NBUF):
        @pl.when(k < nlive)
        def _():
            _start_i(k)

    def _zfill(i, _):
        rows_v[NBUF - 1, i // (D // 16),
               pl.ds((i % (D // 16)) * 16, 16)] = jnp.zeros((16,), jnp.float32)
        return 0

    lax.fori_loop(0, CHUNK * D // 16, _zfill, 0)

    def _zero(k, _):
        pltpu.sync_copy(rows_v.at[NBUF - 1],
                        acc_sh.at[pl.ds(sid * ZROWS + k * CHUNK, CHUNK)])
        return 0

    lax.fori_loop(0, ZFULL, _zero, 0)
    pltpu.sync_copy(rows_v.at[NBUF - 1, pl.ds(0, ZTAIL)],
                    acc_sh.at[pl.ds(sid * ZROWS + ZFULL * CHUNK, ZTAIL)])

    for k in range(NBUF - 1):
        @pl.when(k < nlive)
        def _():
            _wait_i(k)
            _start_g(k)

    plsc.subcore_barrier()

    def _edge(j, _):
        _wait_g(j)
        _start_s(j)

        @pl.when(j >= 1)
        def _():
            _wait_s(j - 1)

        @pl.when(j + 2 < nlive)
        def _():
            _wait_i(j + 2)
            _start_g(j + 2)

        @pl.when(j + 3 < nlive)
        def _():
            _start_i(j + 3)

        return 0

    lax.fori_loop(0, nlive, _edge, 0)

    @pl.when(nlive >= 1)
    def _():
        _wait_s(nlive - 1)

    plsc.subcore_barrier()
    pltpu.sync_copy(acc_sh.at[pl.ds(sid * ZROWS, ZROWS)],
                    out_hbm.at[cid, pl.ds(sid * ZROWS, ZROWS)])


BLK = 1000
GRID = N // BLK


def _disb(cnt):
    deg = cnt[0, :, 0:1] + cnt[1, :, 0:1]
    return jnp.broadcast_to(lax.rsqrt(deg), (BLK, D))


def _in_body(cnt_ref, x_ref, w_ref, hp_ref):
    h = jnp.dot(x_ref[...], w_ref[...], preferred_element_type=jnp.float32)
    hp_ref[...] = _disb(cnt_ref[...]) * h


_in_call = pl.pallas_call(
    _in_body,
    grid=(GRID,),
    in_specs=[
        pl.BlockSpec((NC, BLK, CW), lambda i: (0, i, 0)),
        pl.BlockSpec((BLK, D), lambda i: (i, 0)),
        pl.BlockSpec((D, D), lambda i: (0, 0)),
    ],
    out_specs=pl.BlockSpec((BLK, D), lambda i: (i, 0)),
    out_shape=jax.ShapeDtypeStruct((N, D), jnp.float32),
)


def _mid_body(cnt_ref, p_ref, b_ref, w_ref, hn_ref):
    p = p_ref[...]
    disb = _disb(cnt_ref[...])
    act = jnp.maximum(disb * (p[0] + p[1]) + b_ref[...], 0.0)
    hn_ref[...] = disb * jnp.dot(act, w_ref[...],
                                 preferred_element_type=jnp.float32)


def _mid_act_body(cnt_ref, p_ref, b_ref, w_ref, hn_ref, act_ref):
    p = p_ref[...]
    disb = _disb(cnt_ref[...])
    act = jnp.maximum(disb * (p[0] + p[1]) + b_ref[...], 0.0)
    hn_ref[...] = disb * jnp.dot(act, w_ref[...],
                                 preferred_element_type=jnp.float32)
    act_ref[...] = act


_mid_specs = dict(
    grid=(GRID,),
    in_specs=[
        pl.BlockSpec((NC, BLK, CW), lambda i: (0, i, 0)),
        pl.BlockSpec((NC, BLK, D), lambda i: (0, i, 0)),
        pl.BlockSpec((1, D), lambda i: (0, 0)),
        pl.BlockSpec((D, D), lambda i: (0, 0)),
    ],
)

_mid_call = pl.pallas_call(
    _mid_body,
    out_specs=pl.BlockSpec((BLK, D), lambda i: (i, 0)),
    out_shape=jax.ShapeDtypeStruct((N, D), jnp.float32),
    **_mid_specs,
)

_mid_act_call = pl.pallas_call(
    _mid_act_body,
    out_specs=[
        pl.BlockSpec((BLK, D), lambda i: (i, 0)),
        pl.BlockSpec((BLK, D), lambda i: (i, 0)),
    ],
    out_shape=[
        jax.ShapeDtypeStruct((N, D), jnp.float32),
        jax.ShapeDtypeStruct((N, D), jnp.float32),
    ],
    **_mid_specs,
)


def _out_body(cnt_ref, p_ref, b_ref, xh_ref):
    p = p_ref[...]
    xh_ref[...] = _disb(cnt_ref[...]) * (p[0] + p[1]) + b_ref[...]


_out_call = pl.pallas_call(
    _out_body,
    grid=(GRID,),
    in_specs=[
        pl.BlockSpec((NC, BLK, CW), lambda i: (0, i, 0)),
        pl.BlockSpec((NC, BLK, D), lambda i: (0, i, 0)),
        pl.BlockSpec((1, D), lambda i: (0, 0)),
    ],
    out_specs=pl.BlockSpec((BLK, D), lambda i: (i, 0)),
    out_shape=jax.ShapeDtypeStruct((N, D), jnp.float32),
)


def kernel(x, edge_index, W1, b1, W2, b2, W3, b3, W4, b4):
    iota = jnp.arange(N, dtype=jnp.int32)
    pad = E_PAD - LIVE
    ext_src = jnp.concatenate(
        [edge_index[0], iota, jnp.zeros((pad,), jnp.int32)])
    ext_dst = jnp.concatenate(
        [edge_index[1], iota, jnp.full((pad,), N, jnp.int32)])

    cnt = _count_kernel(ext_dst.reshape(NW * NCHUNK, CHUNK))
    hp1 = _in_call(cnt, x, W1)
    p1 = _prop_kernel(hp1, ext_src, ext_dst)
    hp2 = _mid_call(cnt, p1, b1.reshape(1, D), W2)
    p2 = _prop_kernel(hp2, ext_src, ext_dst)
    hp3, z = _mid_act_call(cnt, p2, b2.reshape(1, D), W3)
    p3 = _prop_kernel(hp3, ext_src, ext_dst)
    hp4 = _mid_call(cnt, p3, b3.reshape(1, D), W4)
    p4 = _prop_kernel(hp4, ext_src, ext_dst)
    x_hat = _out_call(cnt, p4, b4.reshape(1, D))
    return (x_hat, z)


# speedup vs baseline: 1.0325x; 1.0202x over previous
import functools

import jax
import jax.numpy as jnp
from jax import lax
from jax.experimental import pallas as pl
from jax.experimental.pallas import tpu as pltpu
from jax.experimental.pallas import tpu_sc as plsc

N = 10000
D = 128
E = 320000
LIVE = E + N
NC, NS = 2, 16
NW = NC * NS
CHUNK = 128
NCHUNK = 88
EPT = NCHUNK * CHUNK
E_PAD = EPT * NW
ACC_ROWS = 10112
ZROWS = ACC_ROWS // NS
ZFULL = ZROWS // CHUNK
ZTAIL = ZROWS - ZFULL * CHUNK
CW = 16
NBUF = 3
NQ = 4

_sc_mesh = plsc.VectorSubcoreMesh(core_axis_name="c", subcore_axis_name="s")


def _live_chunks(wid):
    n = (LIVE - wid * EPT + CHUNK - 1) // CHUNK
    return jnp.clip(n, 0, NCHUNK)


@functools.partial(
    pl.kernel,
    out_type=jax.ShapeDtypeStruct((NC, ACC_ROWS, CW), jnp.float32),
    mesh=_sc_mesh,
    scratch_types=[
        pltpu.VMEM_SHARED((ACC_ROWS, CW), jnp.float32),
        pltpu.VMEM((NCHUNK, CHUNK), jnp.int32),
        pltpu.VMEM((CHUNK, CW), jnp.float32),
        pltpu.SemaphoreType.DMA,
    ],
)
def _count_kernel(dst_hbm, out_hbm, acc_sh, idx_v, ones_v, csem):
    cid = lax.axis_index("c")
    sid = lax.axis_index("s")
    wid = cid * NS + sid

    def _fill(val):
        def body(i, _):
            ones_v[i, pl.ds(0, 16)] = jnp.zeros((16,), jnp.float32) + val
            return 0
        lax.fori_loop(0, CHUNK, body, 0)

    _fill(0.0)

    def _zero(k, _):
        pltpu.sync_copy(ones_v, acc_sh.at[pl.ds(sid * ZROWS + k * CHUNK, CHUNK)])
        return 0

    lax.fori_loop(0, ZFULL, _zero, 0)
    pltpu.sync_copy(ones_v.at[pl.ds(0, ZTAIL)],
                    acc_sh.at[pl.ds(sid * ZROWS + ZFULL * CHUNK, ZTAIL)])
    _fill(1.0)
    plsc.subcore_barrier()

    pltpu.sync_copy(dst_hbm.at[pl.ds(wid * NCHUNK, NCHUNK)], idx_v)

    nlive = _live_chunks(wid)

    def _scat(j, _):
        pltpu.async_copy(ones_v, acc_sh.at[idx_v.at[j]], csem, add=True)

        @pl.when(j >= 16)
        def _():
            pltpu.make_async_copy(ones_v, acc_sh.at[idx_v.at[0]], csem).wait()

        return 0

    lax.fori_loop(0, nlive, _scat, 0)

    def _cdrain(j, _):
        pltpu.make_async_copy(ones_v, acc_sh.at[idx_v.at[0]], csem).wait()
        return 0

    lax.fori_loop(0, jnp.minimum(nlive, 16), _cdrain, 0)
    plsc.subcore_barrier()
    pltpu.sync_copy(acc_sh.at[pl.ds(sid * ZROWS, ZROWS)],
                    out_hbm.at[cid, pl.ds(sid * ZROWS, ZROWS)])


@functools.partial(
    pl.kernel,
    out_type=jax.ShapeDtypeStruct((NC, ACC_ROWS, D), jnp.float32),
    mesh=_sc_mesh,
    scratch_types=[
        pltpu.VMEM_SHARED((ACC_ROWS, D), jnp.float32),
        pltpu.VMEM((NQ * 2, CHUNK), jnp.int32),
        pltpu.VMEM((NBUF, CHUNK, D), jnp.float32),
        pltpu.SemaphoreType.DMA,
        pltpu.SemaphoreType.DMA,
        pltpu.SemaphoreType.DMA,
    ],
)
def _prop_kernel(hp_hbm, idx_hbm, out_hbm, acc_sh, idx_v, rows_v, isem, gsem,
                 ssem):
    cid = lax.axis_index("c")
    sid = lax.axis_index("s")
    wid = cid * NS + sid

    nlive = _live_chunks(wid)

    def _start_i(jj):
        pltpu.async_copy(idx_hbm.at[wid * NCHUNK + jj],
                         idx_v.at[pl.ds((jj % NQ) * 2, 2)], isem)

    def _wait_i(jj):
        pltpu.make_async_copy(idx_hbm.at[wid * NCHUNK + jj],
                              idx_v.at[pl.ds((jj % NQ) * 2, 2)], isem).wait()

    def _start_g(jj):
        pltpu.async_copy(hp_hbm.at[idx_v.at[(jj % NQ) * 2]],
                         rows_v.at[jj % NBUF], gsem)

    def _wait_g(jj):
        pltpu.make_async_copy(hp_hbm.at[idx_v.at[(jj % NQ) * 2]],
                              rows_v.at[jj % NBUF], gsem).wait()

    def _start_s(jj):
        pltpu.async_copy(rows_v.at[jj % NBUF],
                         acc_sh.at[idx_v.at[(jj % NQ) * 2 + 1]], ssem,
                         add=True)

    def _wait_s(jj):
        pltpu.make_async_copy(rows_v.at[jj % NBUF],
                              acc_sh.at[idx_v.at[(jj % NQ) * 2 + 1]],
                              ssem).wait()

    for k in range(NBUF):
        @pl.when(k < nlive)
        def _():
            _start_i(k)

    def _zfill(i, _):
        rows_v[NBUF - 1, i // (D // 16),
               pl.ds((i % (D // 16)) * 16, 16)] = jnp.zeros((16,), jnp.float32)
        return 0

    lax.fori_loop(0, CHUNK * D // 16, _zfill, 0)

    def _zero(k, _):
        pltpu.sync_copy(rows_v.at[NBUF - 1],
                        acc_sh.at[pl.ds(sid * ZROWS + k * CHUNK, CHUNK)])
        return 0

    lax.fori_loop(0, ZFULL, _zero, 0)
    pltpu.sync_copy(rows_v.at[NBUF - 1, pl.ds(0, ZTAIL)],
                    acc_sh.at[pl.ds(sid * ZROWS + ZFULL * CHUNK, ZTAIL)])

    for k in range(NBUF - 1):
        @pl.when(k < nlive)
        def _():
            _wait_i(k)
            _start_g(k)

    plsc.subcore_barrier()

    def _edge(j, _):
        _wait_g(j)
        _start_s(j)

        @pl.when(j >= 1)
        def _():
            _wait_s(j - 1)

        @pl.when(j + 2 < nlive)
        def _():
            _wait_i(j + 2)
            _start_g(j + 2)

        @pl.when(j + 3 < nlive)
        def _():
            _start_i(j + 3)

        return 0

    lax.fori_loop(0, nlive, _edge, 0)

    @pl.when(nlive >= 1)
    def _():
        _wait_s(nlive - 1)

    plsc.subcore_barrier()
    pltpu.sync_copy(acc_sh.at[pl.ds(sid * ZROWS, ZROWS)],
                    out_hbm.at[cid, pl.ds(sid * ZROWS, ZROWS)])


BLK = 2000
GRID = N // BLK


def _disb(cnt):
    deg = cnt[0, :, 0:1] + cnt[1, :, 0:1]
    return jnp.broadcast_to(lax.rsqrt(deg), (BLK, D))


def _in_body(cnt_ref, x_ref, w_ref, hp_ref):
    h = jnp.dot(x_ref[...], w_ref[...], preferred_element_type=jnp.float32)
    hp_ref[...] = _disb(cnt_ref[...]) * h


_in_call = pl.pallas_call(
    _in_body,
    grid=(GRID,),
    in_specs=[
        pl.BlockSpec((NC, BLK, CW), lambda i: (0, i, 0)),
        pl.BlockSpec((BLK, D), lambda i: (i, 0)),
        pl.BlockSpec((D, D), lambda i: (0, 0)),
    ],
    out_specs=pl.BlockSpec((BLK, D), lambda i: (i, 0)),
    out_shape=jax.ShapeDtypeStruct((N, D), jnp.float32),
)


def _mid_body(cnt_ref, p_ref, b_ref, w_ref, hn_ref):
    p = p_ref[...]
    disb = _disb(cnt_ref[...])
    act = jnp.maximum(disb * (p[0] + p[1]) + b_ref[...], 0.0)
    hn_ref[...] = disb * jnp.dot(act, w_ref[...],
                                 preferred_element_type=jnp.float32)


def _mid_act_body(cnt_ref, p_ref, b_ref, w_ref, hn_ref, act_ref):
    p = p_ref[...]
    disb = _disb(cnt_ref[...])
    act = jnp.maximum(disb * (p[0] + p[1]) + b_ref[...], 0.0)
    hn_ref[...] = disb * jnp.dot(act, w_ref[...],
                                 preferred_element_type=jnp.float32)
    act_ref[...] = act


_mid_specs = dict(
    grid=(GRID,),
    in_specs=[
        pl.BlockSpec((NC, BLK, CW), lambda i: (0, i, 0)),
        pl.BlockSpec((NC, BLK, D), lambda i: (0, i, 0)),
        pl.BlockSpec((1, D), lambda i: (0, 0)),
        pl.BlockSpec((D, D), lambda i: (0, 0)),
    ],
)

_mid_call = pl.pallas_call(
    _mid_body,
    out_specs=pl.BlockSpec((BLK, D), lambda i: (i, 0)),
    out_shape=jax.ShapeDtypeStruct((N, D), jnp.float32),
    **_mid_specs,
)

_mid_act_call = pl.pallas_call(
    _mid_act_body,
    out_specs=[
        pl.BlockSpec((BLK, D), lambda i: (i, 0)),
        pl.BlockSpec((BLK, D), lambda i: (i, 0)),
    ],
    out_shape=[
        jax.ShapeDtypeStruct((N, D), jnp.float32),
        jax.ShapeDtypeStruct((N, D), jnp.float32),
    ],
    **_mid_specs,
)


def _out_body(cnt_ref, p_ref, b_ref, xh_ref):
    p = p_ref[...]
    xh_ref[...] = _disb(cnt_ref[...]) * (p[0] + p[1]) + b_ref[...]


_out_call = pl.pallas_call(
    _out_body,
    grid=(GRID,),
    in_specs=[
        pl.BlockSpec((NC, BLK, CW), lambda i: (0, i, 0)),
        pl.BlockSpec((NC, BLK, D), lambda i: (0, i, 0)),
        pl.BlockSpec((1, D), lambda i: (0, 0)),
    ],
    out_specs=pl.BlockSpec((BLK, D), lambda i: (i, 0)),
    out_shape=jax.ShapeDtypeStruct((N, D), jnp.float32),
)


def kernel(x, edge_index, W1, b1, W2, b2, W3, b3, W4, b4):
    iota = jnp.arange(N, dtype=jnp.int32)
    pad = E_PAD - LIVE
    ext_src = jnp.concatenate(
        [edge_index[0], iota, jnp.zeros((pad,), jnp.int32)])
    ext_dst = jnp.concatenate(
        [edge_index[1], iota, jnp.full((pad,), N, jnp.int32)])
    ext2 = jnp.stack(
        [ext_src.reshape(NW * NCHUNK, CHUNK),
         ext_dst.reshape(NW * NCHUNK, CHUNK)], axis=1)

    cnt = _count_kernel(ext_dst.reshape(NW * NCHUNK, CHUNK))
    hp1 = _in_call(cnt, x, W1)
    p1 = _prop_kernel(hp1, ext2)
    hp2 = _mid_call(cnt, p1, b1.reshape(1, D), W2)
    p2 = _prop_kernel(hp2, ext2)
    hp3, z = _mid_act_call(cnt, p2, b2.reshape(1, D), W3)
    p3 = _prop_kernel(hp3, ext2)
    hp4 = _mid_call(cnt, p3, b3.reshape(1, D), W4)
    p4 = _prop_kernel(hp4, ext2)
    x_hat = _out_call(cnt, p4, b4.reshape(1, D))
    return (x_hat, z)
